# baseline (speedup 1.0000x reference)
import functools

import jax
import jax.numpy as jnp
from jax import lax
from jax.experimental import pallas as pl
from jax.experimental.pallas import tpu as pltpu
from jax.experimental.pallas import tpu_sc as plsc

N_NODES = 10000
IN_FEATS = 256
HIDDEN = 128
NUM_LAYERS = 4
EPS = 1e-5

_NC, _NS = 2, 16
_NW = _NC * _NS
_CH = 40
_NR0 = 192
_NR1 = 64
_STG = 64
_NBUF = 4
_EPAD = _NS * (_NR0 + _NR1) * _CH
_SPLIT = _NS * _NR0 * _CH * 125 // 128
_SLOT0 = _NS * _NR0 * _CH
_IDX_ROWS = _EPAD // _CH
_RPS = 632
_RPS_LAST = N_NODES - (_NS - 1) * _RPS
_ACC_ROWS = N_NODES + 112

_BR = 1000
_GRID = N_NODES // _BR


def _ln(x, g, b):
    m = jnp.mean(x, axis=-1, keepdims=True)
    xm = x - m
    v = jnp.mean(xm * xm, axis=-1, keepdims=True)
    return xm * jax.lax.rsqrt(v + EPS) * g + b


def _pre_body(h_ref, sc_ref, w1_ref, y_ref):
    x = h_ref[...]
    g = sc_ref[0:1, :]
    b = sc_ref[1:2, :]
    xn = _ln(x, g, b)
    y_ref[...] = jnp.dot(xn, w1_ref[...], preferred_element_type=jnp.float32)


def _pre(h, in_g, in_b, w1):
    sc = jnp.stack([in_g, in_b])
    return pl.pallas_call(
        _pre_body,
        grid=(_GRID,),
        in_specs=[
            pl.BlockSpec((_BR, IN_FEATS), lambda r: (r, 0)),
            pl.BlockSpec((2, IN_FEATS), lambda r: (0, 0)),
            pl.BlockSpec((IN_FEATS, HIDDEN), lambda r: (0, 0)),
        ],
        out_specs=pl.BlockSpec((_BR, HIDDEN), lambda r: (r, 0)),
        out_shape=jax.ShapeDtypeStruct((N_NODES, HIDDEN), jnp.float32),
    )(h, sc, w1)


def _mlp_tail(parts_ref, y_ref, sc_ref, w2_ref):
    p = parts_ref[...]
    t = p[0] + p[1] - y_ref[...] + sc_ref[0:1, :]
    t = jnp.maximum(_ln(t, sc_ref[1:2, :], sc_ref[2:3, :]), 0.0)
    t = jnp.dot(t, w2_ref[...], preferred_element_type=jnp.float32)
    t = t + sc_ref[3:4, :]
    t = jnp.maximum(_ln(t, sc_ref[4:5, :], sc_ref[5:6, :]), 0.0)
    t = jnp.maximum(_ln(t, sc_ref[6:7, :], sc_ref[7:8, :]), 0.0)
    return t


def _post_body(parts_ref, y_ref, sc_ref, w2_ref, w1n_ref, res_ref,
               h_ref, yn_ref, *, has_res):
    t = _mlp_tail(parts_ref, y_ref, sc_ref, w2_ref)
    if has_res:
        t = t + res_ref[...]
    h_ref[...] = t
    yn_ref[...] = jnp.dot(t, w1n_ref[...], preferred_element_type=jnp.float32)


def _post(parts, y, scal, w2, w1_next, res):
    has_res = res is not None
    if not has_res:
        res = y
    blk = lambda: pl.BlockSpec((_BR, HIDDEN), lambda r: (r, 0))
    full = lambda d0: pl.BlockSpec((d0, HIDDEN), lambda r: (0, 0))
    return pl.pallas_call(
        functools.partial(_post_body, has_res=has_res),
        grid=(_GRID,),
        in_specs=[
            pl.BlockSpec((2, _BR, HIDDEN), lambda r: (0, r, 0)),
            blk(), full(8), full(HIDDEN), full(HIDDEN), blk(),
        ],
        out_specs=[blk(), blk()],
        out_shape=[
            jax.ShapeDtypeStruct((N_NODES, HIDDEN), jnp.float32),
            jax.ShapeDtypeStruct((N_NODES, HIDDEN), jnp.float32),
        ],
    )(parts, y, scal, w2, w1_next, res)


def _last_body(parts_ref, y_ref, sc_ref, w2_ref, res_ref, out_ref):
    t = _mlp_tail(parts_ref, y_ref, sc_ref, w2_ref)
    t = t + res_ref[...]
    part = jnp.sum(t, axis=0, keepdims=True) * (1.0 / N_NODES)

    @pl.when(pl.program_id(0) == 0)
    def _():
        out_ref[...] = jnp.zeros_like(out_ref)

    out_ref[...] += part


def _last(parts, y, scal, w2, res):
    blk = lambda: pl.BlockSpec((_BR, HIDDEN), lambda r: (r, 0))
    full = lambda d0: pl.BlockSpec((d0, HIDDEN), lambda r: (0, 0))
    return pl.pallas_call(
        _last_body,
        grid=(_GRID,),
        in_specs=[
            pl.BlockSpec((2, _BR, HIDDEN), lambda r: (0, r, 0)),
            blk(), full(8), full(HIDDEN), blk(),
        ],
        out_specs=pl.BlockSpec((1, HIDDEN), lambda r: (0, 0)),
        out_shape=jax.ShapeDtypeStruct((1, HIDDEN), jnp.float32),
    )(parts, y, scal, w2, res)


def _sc_body(y_hbm, src_hbm, dst_hbm, out_hbm, idx_s, idx_d, *scratch):
    bufs = scratch[:_NBUF]
    acc = scratch[_NBUF]
    sems = scratch[_NBUF + 1:]
    c = lax.axis_index("c")
    s = lax.axis_index("s")
    wid = c * _NS + s
    base = s * _RPS
    @pl.when(s < _NS - 1)
    def _():
        pltpu.sync_copy(y_hbm.at[pl.ds(base, _RPS)], acc.at[pl.ds(base, _RPS)])

    @pl.when(s == _NS - 1)
    def _():
        pltpu.sync_copy(y_hbm.at[pl.ds(base, _RPS_LAST)],
                        acc.at[pl.ds(base, _RPS_LAST)])

    plsc.subcore_barrier()

    nrc = jnp.where(c == 0, _NR0, _NR1)
    rowbase = jnp.where(c == 0, s * _NR0, _NS * _NR0 + s * _NR1)

    for stage in range(_NR0 // _STG):
        @pl.when(stage * _STG < nrc)
        def _(stage=stage):
            rb = rowbase + stage * _STG
            pltpu.sync_copy(src_hbm.at[pl.ds(rb, _STG)], idx_s)
            pltpu.sync_copy(dst_hbm.at[pl.ds(rb, _STG)], idx_d)
            for b in range(_NBUF):
                pltpu.async_copy(y_hbm.at[idx_s.at[b]], bufs[b], sems[b])

            @pl.loop(0, _STG, step=_NBUF)
            def _(j):
                for b in range(_NBUF):
                    pltpu.make_async_copy(y_hbm.at[idx_s.at[j + b]],
                                          bufs[b], sems[b]).wait()
                    pltpu.sync_copy(bufs[b], acc.at[idx_d.at[j + b]],
                                    add=True)

                    @pl.when(j + b + _NBUF < _STG)
                    def _():
                        pltpu.async_copy(y_hbm.at[idx_s.at[j + b + _NBUF]],
                                         bufs[b], sems[b])

    plsc.subcore_barrier()

    @pl.when(s < _NS - 1)
    def _():
        pltpu.sync_copy(acc.at[pl.ds(base, _RPS)],
                        out_hbm.at[c].at[pl.ds(base, _RPS)])

    @pl.when(s == _NS - 1)
    def _():
        pltpu.sync_copy(acc.at[pl.ds(base, _RPS_LAST)],
                        out_hbm.at[c].at[pl.ds(base, _RPS_LAST)])


def _sc_agg(y, src_p, dst_p):
    mesh = plsc.VectorSubcoreMesh(core_axis_name="c", subcore_axis_name="s",
                                  num_cores=_NC, num_subcores=_NS)
    f = functools.partial(
        pl.kernel,
        out_type=jax.ShapeDtypeStruct((_NC, N_NODES, HIDDEN), jnp.float32),
        mesh=mesh,
        scratch_types=[
            pltpu.VMEM((_STG, _CH), jnp.int32),
            pltpu.VMEM((_STG, _CH), jnp.int32),
        ]
        + [pltpu.VMEM((_CH, HIDDEN), jnp.float32)] * _NBUF
        + [pltpu.VMEM_SHARED((_ACC_ROWS, HIDDEN), jnp.float32)]
        + [pltpu.SemaphoreType.DMA] * _NBUF,
    )(_sc_body)
    return f(y, src_p, dst_p)


def kernel(h, edge_index, params):
    src = edge_index[0].astype(jnp.int32)
    dst = edge_index[1].astype(jnp.int32)
    pad0 = _SLOT0 - _SPLIT
    pad1 = _EPAD - _SLOT0 - (src.shape[0] - _SPLIT)
    zpad0 = jnp.zeros((pad0,), jnp.int32)
    zpad1 = jnp.zeros((pad1,), jnp.int32)
    src_p = jnp.concatenate([src[:_SPLIT], zpad0, src[_SPLIT:], zpad1])
    dst_p = jnp.concatenate([dst[:_SPLIT], jnp.full((pad0,), N_NODES, jnp.int32),
                             dst[_SPLIT:], jnp.full((pad1,), N_NODES, jnp.int32)])
    src_p = src_p.reshape(_IDX_ROWS, _CH)
    dst_p = dst_p.reshape(_IDX_ROWS, _CH)

    y = _pre(h, params['in_g'], params['in_b'], params['layer0']['W1'])
    hcur = None
    for i in range(NUM_LAYERS):
        p = params[f'layer{i}']
        scal = jnp.stack([p['b1'], p['ln1_g'], p['ln1_b'], p['b2'],
                          p['ln2_g'], p['ln2_b'], p['n_g'], p['n_b']])
        parts = _sc_agg(y, src_p, dst_p)
        if i < NUM_LAYERS - 1:
            w1n = params[f'layer{i + 1}']['W1']
            hnext, ynext = _post(parts, y, scal, p['W2'], w1n, hcur)
            hcur, y = hnext, ynext
        else:
            out = _last(parts, y, scal, p['W2'], hcur)
    return out


# speedup vs baseline: 1.0022x; 1.0022x over previous
import functools

import jax
import jax.numpy as jnp
from jax import lax
from jax.experimental import pallas as pl
from jax.experimental.pallas import tpu as pltpu
from jax.experimental.pallas import tpu_sc as plsc

N_NODES = 10000
IN_FEATS = 256
HIDDEN = 128
NUM_LAYERS = 4
EPS = 1e-5

_NC, _NS = 2, 16
_NW = _NC * _NS
_CH = 40
_KCH = 128
_NBUF = 3
_EPAD = _NW * _KCH * _CH
_IDX_ROWS = _EPAD // _CH
_RPS = 632
_RPS_LAST = N_NODES - (_NS - 1) * _RPS
_ACC_ROWS = N_NODES + 112

_BR = 1000
_GRID = N_NODES // _BR


def _ln(x, g, b):
    m = jnp.mean(x, axis=-1, keepdims=True)
    xm = x - m
    v = jnp.mean(xm * xm, axis=-1, keepdims=True)
    return xm * jax.lax.rsqrt(v + EPS) * g + b


def _pre_body(h_ref, sc_ref, w1_ref, y_ref):
    x = h_ref[...]
    g = sc_ref[0:1, :]
    b = sc_ref[1:2, :]
    xn = _ln(x, g, b)
    y_ref[...] = jnp.dot(xn, w1_ref[...], preferred_element_type=jnp.float32)


def _pre(h, in_g, in_b, w1):
    sc = jnp.stack([in_g, in_b])
    return pl.pallas_call(
        _pre_body,
        grid=(_GRID,),
        in_specs=[
            pl.BlockSpec((_BR, IN_FEATS), lambda r: (r, 0)),
            pl.BlockSpec((2, IN_FEATS), lambda r: (0, 0)),
            pl.BlockSpec((IN_FEATS, HIDDEN), lambda r: (0, 0)),
        ],
        out_specs=pl.BlockSpec((_BR, HIDDEN), lambda r: (r, 0)),
        out_shape=jax.ShapeDtypeStruct((N_NODES, HIDDEN), jnp.float32),
    )(h, sc, w1)


def _mlp_tail(parts_ref, y_ref, sc_ref, w2_ref):
    p = parts_ref[...]
    t = p[0] + p[1] - y_ref[...] + sc_ref[0:1, :]
    t = jnp.maximum(_ln(t, sc_ref[1:2, :], sc_ref[2:3, :]), 0.0)
    t = jnp.dot(t, w2_ref[...], preferred_element_type=jnp.float32)
    t = t + sc_ref[3:4, :]
    t = jnp.maximum(_ln(t, sc_ref[4:5, :], sc_ref[5:6, :]), 0.0)
    t = jnp.maximum(_ln(t, sc_ref[6:7, :], sc_ref[7:8, :]), 0.0)
    return t


def _post_body(parts_ref, y_ref, sc_ref, w2_ref, w1n_ref, res_ref,
               h_ref, yn_ref, *, has_res):
    t = _mlp_tail(parts_ref, y_ref, sc_ref, w2_ref)
    if has_res:
        t = t + res_ref[...]
    h_ref[...] = t
    yn_ref[...] = jnp.dot(t, w1n_ref[...], preferred_element_type=jnp.float32)


def _post(parts, y, scal, w2, w1_next, res):
    has_res = res is not None
    if not has_res:
        res = y
    blk = lambda: pl.BlockSpec((_BR, HIDDEN), lambda r: (r, 0))
    full = lambda d0: pl.BlockSpec((d0, HIDDEN), lambda r: (0, 0))
    return pl.pallas_call(
        functools.partial(_post_body, has_res=has_res),
        grid=(_GRID,),
        in_specs=[
            pl.BlockSpec((2, _BR, HIDDEN), lambda r: (0, r, 0)),
            blk(), full(8), full(HIDDEN), full(HIDDEN), blk(),
        ],
        out_specs=[blk(), blk()],
        out_shape=[
            jax.ShapeDtypeStruct((N_NODES, HIDDEN), jnp.float32),
            jax.ShapeDtypeStruct((N_NODES, HIDDEN), jnp.float32),
        ],
    )(parts, y, scal, w2, w1_next, res)


def _last_body(parts_ref, y_ref, sc_ref, w2_ref, res_ref, out_ref):
    t = _mlp_tail(parts_ref, y_ref, sc_ref, w2_ref)
    t = t + res_ref[...]
    part = jnp.sum(t, axis=0, keepdims=True) * (1.0 / N_NODES)

    @pl.when(pl.program_id(0) == 0)
    def _():
        out_ref[...] = jnp.zeros_like(out_ref)

    out_ref[...] += part


def _last(parts, y, scal, w2, res):
    blk = lambda: pl.BlockSpec((_BR, HIDDEN), lambda r: (r, 0))
    full = lambda d0: pl.BlockSpec((d0, HIDDEN), lambda r: (0, 0))
    return pl.pallas_call(
        _last_body,
        grid=(_GRID,),
        in_specs=[
            pl.BlockSpec((2, _BR, HIDDEN), lambda r: (0, r, 0)),
            blk(), full(8), full(HIDDEN), blk(),
        ],
        out_specs=pl.BlockSpec((1, HIDDEN), lambda r: (0, 0)),
        out_shape=jax.ShapeDtypeStruct((1, HIDDEN), jnp.float32),
    )(parts, y, scal, w2, res)


def _sc_body(y_hbm, src_hbm, dst_hbm, out_hbm, idx_s, idx_d, *scratch):
    bufs = scratch[:_NBUF]
    acc = scratch[_NBUF]
    sems = scratch[_NBUF + 1:]
    c = lax.axis_index("c")
    s = lax.axis_index("s")
    wid = c * _NS + s
    base = s * _RPS
    @pl.when(s < _NS - 1)
    def _():
        pltpu.sync_copy(y_hbm.at[pl.ds(base, _RPS)], acc.at[pl.ds(base, _RPS)])

    @pl.when(s == _NS - 1)
    def _():
        pltpu.sync_copy(y_hbm.at[pl.ds(base, _RPS_LAST)],
                        acc.at[pl.ds(base, _RPS_LAST)])

    pltpu.sync_copy(src_hbm.at[pl.ds(wid * _KCH, _KCH)], idx_s)
    pltpu.sync_copy(dst_hbm.at[pl.ds(wid * _KCH, _KCH)], idx_d)
    plsc.subcore_barrier()

    for b in range(_NBUF):
        pltpu.async_copy(y_hbm.at[idx_s.at[b]], bufs[b], sems[b])

    _MAIN = (_KCH // _NBUF) * _NBUF

    @pl.loop(0, _MAIN, step=_NBUF)
    def _(j):
        for b in range(_NBUF):
            pltpu.make_async_copy(y_hbm.at[idx_s.at[j + b]],
                                  bufs[b], sems[b]).wait()
            pltpu.sync_copy(bufs[b], acc.at[idx_d.at[j + b]], add=True)

            @pl.when(j + b + _NBUF < _KCH)
            def _():
                pltpu.async_copy(y_hbm.at[idx_s.at[j + b + _NBUF]],
                                 bufs[b], sems[b])

    for j in range(_MAIN, _KCH):
        b = j - _MAIN
        pltpu.make_async_copy(y_hbm.at[idx_s.at[j]], bufs[b], sems[b]).wait()
        pltpu.sync_copy(bufs[b], acc.at[idx_d.at[j]], add=True)

    plsc.subcore_barrier()

    @pl.when(s < _NS - 1)
    def _():
        pltpu.sync_copy(acc.at[pl.ds(base, _RPS)],
                        out_hbm.at[c].at[pl.ds(base, _RPS)])

    @pl.when(s == _NS - 1)
    def _():
        pltpu.sync_copy(acc.at[pl.ds(base, _RPS_LAST)],
                        out_hbm.at[c].at[pl.ds(base, _RPS_LAST)])


def _sc_agg(y, src_p, dst_p):
    mesh = plsc.VectorSubcoreMesh(core_axis_name="c", subcore_axis_name="s",
                                  num_cores=_NC, num_subcores=_NS)
    f = functools.partial(
        pl.kernel,
        out_type=jax.ShapeDtypeStruct((_NC, N_NODES, HIDDEN), jnp.float32),
        mesh=mesh,
        scratch_types=[
            pltpu.VMEM((_KCH, _CH), jnp.int32),
            pltpu.VMEM((_KCH, _CH), jnp.int32),
        ]
        + [pltpu.VMEM((_CH, HIDDEN), jnp.float32)] * _NBUF
        + [pltpu.VMEM_SHARED((_ACC_ROWS, HIDDEN), jnp.float32)]
        + [pltpu.SemaphoreType.DMA] * _NBUF,
    )(_sc_body)
    return f(y, src_p, dst_p)


def kernel(h, edge_index, params):
    src = edge_index[0].astype(jnp.int32)
    dst = edge_index[1].astype(jnp.int32)
    npad = _EPAD - src.shape[0]
    src_p = jnp.concatenate([src, jnp.zeros((npad,), jnp.int32)])
    dst_p = jnp.concatenate([dst, jnp.full((npad,), N_NODES, jnp.int32)])
    src_p = src_p.reshape(_IDX_ROWS, _CH)
    dst_p = dst_p.reshape(_IDX_ROWS, _CH)

    y = _pre(h, params['in_g'], params['in_b'], params['layer0']['W1'])
    hcur = None
    for i in range(NUM_LAYERS):
        p = params[f'layer{i}']
        scal = jnp.stack([p['b1'], p['ln1_g'], p['ln1_b'], p['b2'],
                          p['ln2_g'], p['ln2_b'], p['n_g'], p['n_b']])
        parts = _sc_agg(y, src_p, dst_p)
        if i < NUM_LAYERS - 1:
            w1n = params[f'layer{i + 1}']['W1']
            hnext, ynext = _post(parts, y, scal, p['W2'], w1n, hcur)
            hcur, y = hnext, ynext
        else:
            out = _last(parts, y, scal, p['W2'], hcur)
    return out


# speedup vs baseline: 1.0141x; 1.0119x over previous
import functools

import jax
import jax.numpy as jnp
from jax import lax
from jax.experimental import pallas as pl
from jax.experimental.pallas import tpu as pltpu
from jax.experimental.pallas import tpu_sc as plsc

N_NODES = 10000
IN_FEATS = 256
HIDDEN = 128
NUM_LAYERS = 4
EPS = 1e-5

_NC, _NS = 2, 16
_NW = _NC * _NS
_CH = 40
_KCH = 128
_NBUF = 3
_EPAD = _NW * _KCH * _CH
_IDX_ROWS = _EPAD // _CH
_RPS = 632
_RPS_LAST = N_NODES - (_NS - 1) * _RPS
_ACC_ROWS = N_NODES + 112

_BR = 2000
_GRID = N_NODES // _BR


def _ln(x, g, b):
    m = jnp.mean(x, axis=-1, keepdims=True)
    xm = x - m
    v = jnp.mean(xm * xm, axis=-1, keepdims=True)
    return xm * jax.lax.rsqrt(v + EPS) * g + b


def _pre_body(h_ref, sc_ref, w1_ref, y_ref):
    x = h_ref[...]
    g = sc_ref[0:1, :]
    b = sc_ref[1:2, :]
    xn = _ln(x, g, b)
    y_ref[...] = jnp.dot(xn, w1_ref[...], preferred_element_type=jnp.float32)


def _pre(h, in_g, in_b, w1):
    sc = jnp.stack([in_g, in_b])
    return pl.pallas_call(
        _pre_body,
        grid=(_GRID,),
        in_specs=[
            pl.BlockSpec((_BR, IN_FEATS), lambda r: (r, 0)),
            pl.BlockSpec((2, IN_FEATS), lambda r: (0, 0)),
            pl.BlockSpec((IN_FEATS, HIDDEN), lambda r: (0, 0)),
        ],
        out_specs=pl.BlockSpec((_BR, HIDDEN), lambda r: (r, 0)),
        out_shape=jax.ShapeDtypeStruct((N_NODES, HIDDEN), jnp.float32),
    )(h, sc, w1)


def _mlp_tail(parts_ref, y_ref, sc_ref, w2_ref):
    p = parts_ref[...]
    t = p[0] + p[1] - y_ref[...] + sc_ref[0:1, :]
    t = jnp.maximum(_ln(t, sc_ref[1:2, :], sc_ref[2:3, :]), 0.0)
    t = jnp.dot(t, w2_ref[...], preferred_element_type=jnp.float32)
    t = t + sc_ref[3:4, :]
    t = jnp.maximum(_ln(t, sc_ref[4:5, :], sc_ref[5:6, :]), 0.0)
    t = jnp.maximum(_ln(t, sc_ref[6:7, :], sc_ref[7:8, :]), 0.0)
    return t


def _post_body(parts_ref, y_ref, sc_ref, w2_ref, w1n_ref, res_ref,
               h_ref, yn_ref, *, has_res):
    t = _mlp_tail(parts_ref, y_ref, sc_ref, w2_ref)
    if has_res:
        t = t + res_ref[...]
    h_ref[...] = t
    yn_ref[...] = jnp.dot(t, w1n_ref[...], preferred_element_type=jnp.float32)


def _post(parts, y, scal, w2, w1_next, res):
    has_res = res is not None
    if not has_res:
        res = y
    blk = lambda: pl.BlockSpec((_BR, HIDDEN), lambda r: (r, 0))
    full = lambda d0: pl.BlockSpec((d0, HIDDEN), lambda r: (0, 0))
    return pl.pallas_call(
        functools.partial(_post_body, has_res=has_res),
        grid=(_GRID,),
        in_specs=[
            pl.BlockSpec((2, _BR, HIDDEN), lambda r: (0, r, 0)),
            blk(), full(8), full(HIDDEN), full(HIDDEN), blk(),
        ],
        out_specs=[blk(), blk()],
        out_shape=[
            jax.ShapeDtypeStruct((N_NODES, HIDDEN), jnp.float32),
            jax.ShapeDtypeStruct((N_NODES, HIDDEN), jnp.float32),
        ],
    )(parts, y, scal, w2, w1_next, res)


def _last_body(parts_ref, y_ref, sc_ref, w2_ref, res_ref, out_ref):
    t = _mlp_tail(parts_ref, y_ref, sc_ref, w2_ref)
    t = t + res_ref[...]
    part = jnp.sum(t, axis=0, keepdims=True) * (1.0 / N_NODES)

    @pl.when(pl.program_id(0) == 0)
    def _():
        out_ref[...] = jnp.zeros_like(out_ref)

    out_ref[...] += part


def _last(parts, y, scal, w2, res):
    blk = lambda: pl.BlockSpec((_BR, HIDDEN), lambda r: (r, 0))
    full = lambda d0: pl.BlockSpec((d0, HIDDEN), lambda r: (0, 0))
    return pl.pallas_call(
        _last_body,
        grid=(_GRID,),
        in_specs=[
            pl.BlockSpec((2, _BR, HIDDEN), lambda r: (0, r, 0)),
            blk(), full(8), full(HIDDEN), blk(),
        ],
        out_specs=pl.BlockSpec((1, HIDDEN), lambda r: (0, 0)),
        out_shape=jax.ShapeDtypeStruct((1, HIDDEN), jnp.float32),
    )(parts, y, scal, w2, res)


def _sc_body(y_hbm, src_hbm, dst_hbm, out_hbm, idx_s, idx_d, *scratch):
    bufs = scratch[:_NBUF]
    acc = scratch[_NBUF]
    sems = scratch[_NBUF + 1:]
    c = lax.axis_index("c")
    s = lax.axis_index("s")
    wid = c * _NS + s
    base = s * _RPS
    @pl.when(s < _NS - 1)
    def _():
        pltpu.sync_copy(y_hbm.at[pl.ds(base, _RPS)], acc.at[pl.ds(base, _RPS)])

    @pl.when(s == _NS - 1)
    def _():
        pltpu.sync_copy(y_hbm.at[pl.ds(base, _RPS_LAST)],
                        acc.at[pl.ds(base, _RPS_LAST)])

    pltpu.sync_copy(src_hbm.at[pl.ds(wid * _KCH, _KCH)], idx_s)
    pltpu.sync_copy(dst_hbm.at[pl.ds(wid * _KCH, _KCH)], idx_d)
    plsc.subcore_barrier()

    for b in range(_NBUF):
        pltpu.async_copy(y_hbm.at[idx_s.at[b]], bufs[b], sems[b])

    _MAIN = (_KCH // _NBUF) * _NBUF

    @pl.loop(0, _MAIN, step=_NBUF)
    def _(j):
        for b in range(_NBUF):
            pltpu.make_async_copy(y_hbm.at[idx_s.at[j + b]],
                                  bufs[b], sems[b]).wait()
            pltpu.sync_copy(bufs[b], acc.at[idx_d.at[j + b]], add=True)

            @pl.when(j + b + _NBUF < _KCH)
            def _():
                pltpu.async_copy(y_hbm.at[idx_s.at[j + b + _NBUF]],
                                 bufs[b], sems[b])

    for j in range(_MAIN, _KCH):
        b = j - _MAIN
        pltpu.make_async_copy(y_hbm.at[idx_s.at[j]], bufs[b], sems[b]).wait()
        pltpu.sync_copy(bufs[b], acc.at[idx_d.at[j]], add=True)

    plsc.subcore_barrier()

    @pl.when(s < _NS - 1)
    def _():
        pltpu.sync_copy(acc.at[pl.ds(base, _RPS)],
                        out_hbm.at[c].at[pl.ds(base, _RPS)])

    @pl.when(s == _NS - 1)
    def _():
        pltpu.sync_copy(acc.at[pl.ds(base, _RPS_LAST)],
                        out_hbm.at[c].at[pl.ds(base, _RPS_LAST)])


def _sc_agg(y, src_p, dst_p):
    mesh = plsc.VectorSubcoreMesh(core_axis_name="c", subcore_axis_name="s",
                                  num_cores=_NC, num_subcores=_NS)
    f = functools.partial(
        pl.kernel,
        out_type=jax.ShapeDtypeStruct((_NC, N_NODES, HIDDEN), jnp.float32),
        mesh=mesh,
        scratch_types=[
            pltpu.VMEM((_KCH, _CH), jnp.int32),
            pltpu.VMEM((_KCH, _CH), jnp.int32),
        ]
        + [pltpu.VMEM((_CH, HIDDEN), jnp.float32)] * _NBUF
        + [pltpu.VMEM_SHARED((_ACC_ROWS, HIDDEN), jnp.float32)]
        + [pltpu.SemaphoreType.DMA] * _NBUF,
    )(_sc_body)
    return f(y, src_p, dst_p)


def kernel(h, edge_index, params):
    src = edge_index[0].astype(jnp.int32)
    dst = edge_index[1].astype(jnp.int32)
    npad = _EPAD - src.shape[0]
    src_p = jnp.concatenate([src, jnp.zeros((npad,), jnp.int32)])
    dst_p = jnp.concatenate([dst, jnp.full((npad,), N_NODES, jnp.int32)])
    src_p = src_p.reshape(_IDX_ROWS, _CH)
    dst_p = dst_p.reshape(_IDX_ROWS, _CH)

    y = _pre(h, params['in_g'], params['in_b'], params['layer0']['W1'])
    hcur = None
    for i in range(NUM_LAYERS):
        p = params[f'layer{i}']
        scal = jnp.stack([p['b1'], p['ln1_g'], p['ln1_b'], p['b2'],
                          p['ln2_g'], p['ln2_b'], p['n_g'], p['n_b']])
        parts = _sc_agg(y, src_p, dst_p)
        if i < NUM_LAYERS - 1:
            w1n = params[f'layer{i + 1}']['W1']
            hnext, ynext = _post(parts, y, scal, p['W2'], w1n, hcur)
            hcur, y = hnext, ynext
        else:
            out = _last(parts, y, scal, p['W2'], hcur)
    return out
